# Initial kernel scaffold; baseline (speedup 1.0000x reference)
import jax
import jax.numpy as jnp
from jax.experimental import pallas as pl


def kernel(num_features, cat_features, emb_tables, W_cat, b_cat, ln_g, ln_b, bn0_g, bn0_b, W_n1, b_n1, bn1_g, bn1_b, W_n2, b_n2, bn2_g, bn2_b, W_h1, b_h1, bnh_g, bnh_b, W_h2, b_h2):
    raise NotImplementedError("write your pallas kernel here")


# speedup vs baseline: 6.2773x; 6.2773x over previous
import functools

import jax
import jax.numpy as jnp
from jax.experimental import pallas as pl
from jax.experimental.pallas import tpu as pltpu
from jax.experimental.pallas import tpu_sc as plsc

B = 16384
N_CAT = 26
VOCAB = 100000
EMB = 32
N_NUM = 13
CAT_H = 128
NUM_H = 128
HEAD_H = CAT_H + NUM_H
OUT = 64
EPS = 1e-5

N_TOTAL = B * N_CAT
WIDE = 128
GRP = WIDE // EMB
V_WIDE = N_CAT * VOCAB // GRP
SC_CORES = 2
SC_SUBCORES = 16
NW = SC_CORES * SC_SUBCORES
PER_W = N_TOTAL // NW
CHUNK = 512
N_CHUNKS = PER_W // CHUNK
BLK = 1024
NB = B // BLK


def _leaky(x):
    return jnp.where(x >= 0, x, 0.01 * x)


def _sc_gather(tables_wide, flat_idx):
    mesh = plsc.VectorSubcoreMesh(core_axis_name="c", subcore_axis_name="s")

    @functools.partial(
        pl.kernel, mesh=mesh,
        out_type=jax.ShapeDtypeStruct((N_TOTAL, WIDE), tables_wide.dtype),
        scratch_types=[
            pltpu.VMEM((CHUNK,), jnp.int32),
            pltpu.VMEM((CHUNK, WIDE), jnp.float32),
            pltpu.SemaphoreType.DMA,
        ],
    )
    def kern(table_hbm, idx_hbm, out_hbm, idx_v, rows_v, sem):
        wid = jax.lax.axis_index("s") * SC_CORES + jax.lax.axis_index("c")
        base = wid * PER_W

        @pl.loop(0, N_CHUNKS)
        def _(ci):
            off = base + ci * CHUNK
            pltpu.sync_copy(idx_hbm.at[pl.ds(off, CHUNK)], idx_v)
            pltpu.async_copy(table_hbm.at[idx_v], rows_v, sem).wait()
            pltpu.sync_copy(rows_v, out_hbm.at[pl.ds(off, CHUNK)])

    return kern(tables_wide, flat_idx)


def _num_tower_body(x_ref, w1_ref, b1_ref, w2_ref, b2_ref,
                    bn0g_ref, bn0b_ref, bn1g_ref, bn1b_ref,
                    bn2g_ref, bn2b_ref, o_ref):
    x = x_ref[...]
    mu = jnp.mean(x, axis=0, keepdims=True)
    var = jnp.mean((x - mu) ** 2, axis=0, keepdims=True)
    h = bn0g_ref[...] * (x - mu) / jnp.sqrt(var + EPS) + bn0b_ref[...]
    h = jnp.dot(h, w1_ref[...], preferred_element_type=jnp.float32) + b1_ref[...]
    mu = jnp.mean(h, axis=0, keepdims=True)
    var = jnp.mean((h - mu) ** 2, axis=0, keepdims=True)
    h = _leaky(bn1g_ref[...] * (h - mu) / jnp.sqrt(var + EPS) + bn1b_ref[...])
    h = jnp.dot(h, w2_ref[...], preferred_element_type=jnp.float32) + b2_ref[...]
    mu = jnp.mean(h, axis=0, keepdims=True)
    var = jnp.mean((h - mu) ** 2, axis=0, keepdims=True)
    o_ref[...] = _leaky(bn2g_ref[...] * (h - mu) / jnp.sqrt(var + EPS)
                        + bn2b_ref[...])


def _mid_body(g_ref, m_ref, n_ref, wwide_ref, bcat_ref, lng_ref, lnb_ref,
              wh1_ref, bh1_ref, e1_ref, stats_ref, msk_scratch):
    i = pl.program_id(0)
    grp_id = jax.lax.broadcasted_iota(jnp.int32, (1, WIDE), 1) // EMB
    for f in range(N_CAT):
        sl = slice(f * WIDE, (f + 1) * WIDE)
        mf = m_ref[:, f:f + 1]
        msk = (mf == grp_id).astype(jnp.float32)
        msk_scratch[:, sl] = (g_ref[:, sl] * msk).astype(jnp.bfloat16)
    c = jnp.dot(msk_scratch[...], wwide_ref[...],
                preferred_element_type=jnp.float32) + bcat_ref[...]
    mu = jnp.mean(c, axis=-1, keepdims=True)
    var = jnp.mean((c - mu) ** 2, axis=-1, keepdims=True)
    c = lng_ref[...] * (c - mu) / jnp.sqrt(var + EPS) + lnb_ref[...]
    e1 = (jnp.dot(n_ref[...], wh1_ref[0:NUM_H, :],
                  preferred_element_type=jnp.float32)
          + jnp.dot(c, wh1_ref[NUM_H:HEAD_H, :],
                    preferred_element_type=jnp.float32)
          + bh1_ref[...])
    e1_ref[...] = e1
    s = jnp.sum(e1, axis=0, keepdims=True)
    s2 = jnp.sum(e1 * e1, axis=0, keepdims=True)
    st = jnp.concatenate([s, s2], axis=0)

    @pl.when(i == 0)
    def _():
        stats_ref[...] = st

    @pl.when(i > 0)
    def _():
        stats_ref[...] += st


def _head_body(e1_ref, stats_ref, wh2_ref, bh2_ref, bnhg_ref, bnhb_ref, o_ref):
    st = stats_ref[...]
    mu = st[0:1, :] * (1.0 / B)
    var = st[1:2, :] * (1.0 / B) - mu * mu
    e = _leaky(bnhg_ref[...] * (e1_ref[...] - mu) / jnp.sqrt(var + EPS)
               + bnhb_ref[...])
    e = jnp.dot(e, wh2_ref[...], preferred_element_type=jnp.float32) + bh2_ref[...]
    o_ref[...] = e / jnp.sqrt(jnp.sum(e * e, axis=-1, keepdims=True))


def kernel(num_features, cat_features, emb_tables, W_cat, b_cat, ln_g, ln_b,
           bn0_g, bn0_b, W_n1, b_n1, bn1_g, bn1_b, W_n2, b_n2, bn2_g, bn2_b,
           W_h1, b_h1, bnh_g, bnh_b, W_h2, b_h2):
    f32 = jnp.float32
    r2 = lambda v: v.reshape(1, -1)

    tables_wide = emb_tables.reshape(V_WIDE, WIDE)
    flat_idx = (cat_features.astype(jnp.int32)
                + (jnp.arange(N_CAT, dtype=jnp.int32) * VOCAB)[None, :])
    widx = flat_idx // GRP
    grp = flat_idx % GRP
    gathered = _sc_gather(tables_wide, widx.reshape(N_TOTAL))
    gathered = gathered.reshape(B, N_CAT * WIDE)

    W_wide = jnp.broadcast_to(
        W_cat.reshape(N_CAT, 1, EMB, CAT_H), (N_CAT, GRP, EMB, CAT_H)
    ).reshape(N_CAT * WIDE, CAT_H).astype(jnp.bfloat16)

    num_embs = pl.pallas_call(
        _num_tower_body,
        out_shape=jax.ShapeDtypeStruct((B, NUM_H), f32),
    )(num_features, W_n1, r2(b_n1), W_n2, r2(b_n2),
      r2(bn0_g), r2(bn0_b), r2(bn1_g), r2(bn1_b), r2(bn2_g), r2(bn2_b))

    row_blk = lambda i: (i, 0)
    whole = lambda i: (0, 0)
    e1, stats = pl.pallas_call(
        _mid_body,
        grid=(NB,),
        in_specs=[
            pl.BlockSpec((BLK, N_CAT * WIDE), row_blk),
            pl.BlockSpec((BLK, N_CAT), row_blk),
            pl.BlockSpec((BLK, NUM_H), row_blk),
            pl.BlockSpec((N_CAT * WIDE, CAT_H), whole),
            pl.BlockSpec((1, CAT_H), whole),
            pl.BlockSpec((1, CAT_H), whole),
            pl.BlockSpec((1, CAT_H), whole),
            pl.BlockSpec((HEAD_H, HEAD_H), whole),
            pl.BlockSpec((1, HEAD_H), whole),
        ],
        out_specs=[
            pl.BlockSpec((BLK, HEAD_H), row_blk),
            pl.BlockSpec((2, HEAD_H), whole),
        ],
        out_shape=[
            jax.ShapeDtypeStruct((B, HEAD_H), f32),
            jax.ShapeDtypeStruct((2, HEAD_H), f32),
        ],
        scratch_shapes=[pltpu.VMEM((BLK, N_CAT * WIDE), jnp.bfloat16)],
    )(gathered, grp, num_embs, W_wide, r2(b_cat), r2(ln_g), r2(ln_b),
      W_h1, r2(b_h1))

    out = pl.pallas_call(
        _head_body,
        grid=(NB,),
        in_specs=[
            pl.BlockSpec((BLK, HEAD_H), row_blk),
            pl.BlockSpec((2, HEAD_H), whole),
            pl.BlockSpec((HEAD_H, OUT), whole),
            pl.BlockSpec((1, OUT), whole),
            pl.BlockSpec((1, HEAD_H), whole),
            pl.BlockSpec((1, HEAD_H), whole),
        ],
        out_specs=pl.BlockSpec((BLK, OUT), row_blk),
        out_shape=jax.ShapeDtypeStruct((B, OUT), f32),
    )(e1, stats, W_h2, r2(b_h2), r2(bnh_g), r2(bnh_b))

    return out


# speedup vs baseline: 7.7608x; 1.2363x over previous
import functools

import jax
import jax.numpy as jnp
from jax.experimental import pallas as pl
from jax.experimental.pallas import tpu as pltpu
from jax.experimental.pallas import tpu_sc as plsc

B = 16384
N_CAT = 26
VOCAB = 100000
EMB = 32
N_NUM = 13
CAT_H = 128
NUM_H = 128
HEAD_H = CAT_H + NUM_H
OUT = 64
EPS = 1e-5

N_TOTAL = B * N_CAT
SC_CORES = 2
SC_SUBCORES = 16
NW = SC_CORES * SC_SUBCORES
PER_W = N_TOTAL // NW
CHUNK = 512
N_CHUNKS = PER_W // CHUNK
BLK = 1024
NB = B // BLK


def _leaky(x):
    return jnp.where(x >= 0, x, 0.01 * x)


def _sc_gather(emb_tables, flat_idx):
    mesh = plsc.VectorSubcoreMesh(core_axis_name="c", subcore_axis_name="s")

    @functools.partial(
        pl.kernel, mesh=mesh,
        out_type=jax.ShapeDtypeStruct((N_TOTAL, EMB), emb_tables.dtype),
        scratch_types=[
            pltpu.VMEM((CHUNK,), jnp.int32),
            pltpu.VMEM((CHUNK, EMB), jnp.float32),
            pltpu.SemaphoreType.DMA,
        ],
        compiler_params=pltpu.CompilerParams(use_tc_tiling_on_sc=False),
    )
    def kern(table_hbm, idx_hbm, out_hbm, idx_v, rows_v, sem):
        wid = jax.lax.axis_index("s") * SC_CORES + jax.lax.axis_index("c")
        base = wid * PER_W

        @pl.loop(0, N_CHUNKS)
        def _(ci):
            off = base + ci * CHUNK
            pltpu.sync_copy(idx_hbm.at[pl.ds(off, CHUNK)], idx_v)
            pltpu.async_copy(table_hbm.at[idx_v], rows_v, sem).wait()
            pltpu.sync_copy(rows_v, out_hbm.at[pl.ds(off, CHUNK)])

    return kern(emb_tables.reshape(N_CAT * VOCAB, EMB), flat_idx)


def _num_tower_body(x_ref, w1_ref, b1_ref, w2_ref, b2_ref,
                    bn0g_ref, bn0b_ref, bn1g_ref, bn1b_ref,
                    bn2g_ref, bn2b_ref, o_ref):
    x = x_ref[...]
    mu = jnp.mean(x, axis=0, keepdims=True)
    var = jnp.mean((x - mu) ** 2, axis=0, keepdims=True)
    h = bn0g_ref[...] * (x - mu) / jnp.sqrt(var + EPS) + bn0b_ref[...]
    h = jnp.dot(h, w1_ref[...], preferred_element_type=jnp.float32) + b1_ref[...]
    mu = jnp.mean(h, axis=0, keepdims=True)
    var = jnp.mean((h - mu) ** 2, axis=0, keepdims=True)
    h = _leaky(bn1g_ref[...] * (h - mu) / jnp.sqrt(var + EPS) + bn1b_ref[...])
    h = jnp.dot(h, w2_ref[...], preferred_element_type=jnp.float32) + b2_ref[...]
    mu = jnp.mean(h, axis=0, keepdims=True)
    var = jnp.mean((h - mu) ** 2, axis=0, keepdims=True)
    o_ref[...] = _leaky(bn2g_ref[...] * (h - mu) / jnp.sqrt(var + EPS)
                        + bn2b_ref[...])


def _mid_body(g_ref, n_ref, wcat_ref, bcat_ref, lng_ref, lnb_ref,
              wh1_ref, bh1_ref, e1_ref, stats_ref):
    i = pl.program_id(0)
    c = jnp.dot(g_ref[...], wcat_ref[...],
                preferred_element_type=jnp.float32) + bcat_ref[...]
    mu = jnp.mean(c, axis=-1, keepdims=True)
    var = jnp.mean((c - mu) ** 2, axis=-1, keepdims=True)
    c = lng_ref[...] * (c - mu) / jnp.sqrt(var + EPS) + lnb_ref[...]
    e1 = (jnp.dot(n_ref[...], wh1_ref[0:NUM_H, :],
                  preferred_element_type=jnp.float32)
          + jnp.dot(c, wh1_ref[NUM_H:HEAD_H, :],
                    preferred_element_type=jnp.float32)
          + bh1_ref[...])
    e1_ref[...] = e1
    s = jnp.sum(e1, axis=0, keepdims=True)
    s2 = jnp.sum(e1 * e1, axis=0, keepdims=True)
    st = jnp.concatenate([s, s2], axis=0)

    @pl.when(i == 0)
    def _():
        stats_ref[...] = st

    @pl.when(i > 0)
    def _():
        stats_ref[...] += st


def _head_body(e1_ref, stats_ref, wh2_ref, bh2_ref, bnhg_ref, bnhb_ref, o_ref):
    st = stats_ref[...]
    mu = st[0:1, :] * (1.0 / B)
    var = st[1:2, :] * (1.0 / B) - mu * mu
    e = _leaky(bnhg_ref[...] * (e1_ref[...] - mu) / jnp.sqrt(var + EPS)
               + bnhb_ref[...])
    e = jnp.dot(e, wh2_ref[...], preferred_element_type=jnp.float32) + bh2_ref[...]
    o_ref[...] = e / jnp.sqrt(jnp.sum(e * e, axis=-1, keepdims=True))


def kernel(num_features, cat_features, emb_tables, W_cat, b_cat, ln_g, ln_b,
           bn0_g, bn0_b, W_n1, b_n1, bn1_g, bn1_b, W_n2, b_n2, bn2_g, bn2_b,
           W_h1, b_h1, bnh_g, bnh_b, W_h2, b_h2):
    f32 = jnp.float32
    r2 = lambda v: v.reshape(1, -1)

    flat_idx = (cat_features.astype(jnp.int32)
                + (jnp.arange(N_CAT, dtype=jnp.int32) * VOCAB)[None, :])
    gathered = _sc_gather(emb_tables, flat_idx.reshape(N_TOTAL))
    gathered = gathered.reshape(B, N_CAT * EMB)

    num_embs = pl.pallas_call(
        _num_tower_body,
        out_shape=jax.ShapeDtypeStruct((B, NUM_H), f32),
    )(num_features, W_n1, r2(b_n1), W_n2, r2(b_n2),
      r2(bn0_g), r2(bn0_b), r2(bn1_g), r2(bn1_b), r2(bn2_g), r2(bn2_b))

    row_blk = lambda i: (i, 0)
    whole = lambda i: (0, 0)
    e1, stats = pl.pallas_call(
        _mid_body,
        grid=(NB,),
        in_specs=[
            pl.BlockSpec((BLK, N_CAT * EMB), row_blk),
            pl.BlockSpec((BLK, NUM_H), row_blk),
            pl.BlockSpec((N_CAT * EMB, CAT_H), whole),
            pl.BlockSpec((1, CAT_H), whole),
            pl.BlockSpec((1, CAT_H), whole),
            pl.BlockSpec((1, CAT_H), whole),
            pl.BlockSpec((HEAD_H, HEAD_H), whole),
            pl.BlockSpec((1, HEAD_H), whole),
        ],
        out_specs=[
            pl.BlockSpec((BLK, HEAD_H), row_blk),
            pl.BlockSpec((2, HEAD_H), whole),
        ],
        out_shape=[
            jax.ShapeDtypeStruct((B, HEAD_H), f32),
            jax.ShapeDtypeStruct((2, HEAD_H), f32),
        ],
    )(gathered, num_embs, W_cat, r2(b_cat), r2(ln_g), r2(ln_b),
      W_h1, r2(b_h1))

    out = pl.pallas_call(
        _head_body,
        grid=(NB,),
        in_specs=[
            pl.BlockSpec((BLK, HEAD_H), row_blk),
            pl.BlockSpec((2, HEAD_H), whole),
            pl.BlockSpec((HEAD_H, OUT), whole),
            pl.BlockSpec((1, OUT), whole),
            pl.BlockSpec((1, HEAD_H), whole),
            pl.BlockSpec((1, HEAD_H), whole),
        ],
        out_specs=pl.BlockSpec((BLK, OUT), row_blk),
        out_shape=jax.ShapeDtypeStruct((B, OUT), f32),
    )(e1, stats, W_h2, r2(b_h2), r2(bnh_g), r2(bnh_b))

    return out
